# baseline (speedup 1.0000x reference)
import jax
import jax.numpy as jnp
from jax.experimental import pallas as pl
from jax.experimental.pallas import tpu as pltpu


def _fused_body(x_ref, w_ref, a1_ref, a2_ref, b_ref, o_ref, s_ref):
    @pl.when(pl.program_id(0) == 0)
    def _():
        s_ref[...] = jnp.dot(
            x_ref[...].astype(jnp.bfloat16),
            w_ref[...].astype(jnp.bfloat16),
            preferred_element_type=jnp.float32,
        ).astype(jnp.bfloat16)

    h = a1_ref.shape[0]
    o_ref[0:h, :] = (
        jnp.dot(
            a1_ref[...].astype(jnp.bfloat16),
            s_ref[...],
            preferred_element_type=jnp.float32,
        )
        + b_ref[...]
    )
    o_ref[h : 2 * h, :] = (
        jnp.dot(
            a2_ref[...].astype(jnp.bfloat16),
            s_ref[...],
            preferred_element_type=jnp.float32,
        )
        + b_ref[...]
    )


def _row_block(n):
    best = 16
    for bm in range(16, 513, 16):
        if n % bm == 0:
            best = bm
    return best


def kernel(input_features, adj, W, b):
    n, d_in = input_features.shape
    d_out = W.shape[1]
    bm = _row_block(n)
    half = bm // 2
    out = pl.pallas_call(
        _fused_body,
        grid=(n // bm,),
        in_specs=[
            pl.BlockSpec((n, d_in), lambda i: (0, 0)),
            pl.BlockSpec((d_in, d_out), lambda i: (0, 0)),
            pl.BlockSpec((half, n), lambda i: (2 * i, 0)),
            pl.BlockSpec((half, n), lambda i: (2 * i + 1, 0)),
            pl.BlockSpec((1, d_out), lambda i: (0, 0)),
        ],
        out_specs=pl.BlockSpec((bm, d_out), lambda i: (i, 0)),
        out_shape=jax.ShapeDtypeStruct((n, d_out), jnp.float32),
        scratch_shapes=[pltpu.VMEM((n, d_out), jnp.bfloat16)],
    )(input_features, W, adj, adj, b.reshape(1, d_out))
    return out
